# baseline (speedup 1.0000x reference)
import functools
import math

import jax
import jax.numpy as jnp
from jax import lax
from jax.experimental import pallas as pl
from jax.experimental.pallas import tpu as pltpu
from jax.experimental.pallas import tpu_sc as plsc

_SCALE = math.sqrt(128.0)
_NCHUNKS = 2


def _gather3d(table, idx, n_total, s, row_base, n_rows):
    d = table.shape[1]
    nc, ns = 2, 16
    nw = nc * ns
    rpt = n_rows // nw
    c = 8
    nchunk = rpt // c
    w = c * s

    mesh = plsc.VectorSubcoreMesh(core_axis_name="c", subcore_axis_name="s")

    @functools.partial(
        pl.kernel,
        out_type=jax.ShapeDtypeStruct((n_rows, s, d), jnp.float32),
        mesh=mesh,
        scratch_types=[
            pltpu.VMEM((rpt * s,), jnp.int32),
            pltpu.VMEM((w, d), jnp.float32),
            pltpu.VMEM((w, d), jnp.float32),
            pltpu.SemaphoreType.DMA,
            pltpu.SemaphoreType.DMA,
            pltpu.SemaphoreType.DMA,
            pltpu.SemaphoreType.DMA,
        ],
    )
    def k(table_hbm, i_hbm, o_hbm, idx_v, buf0, buf1, g0, g1, o0, o1):
        wid = lax.axis_index("s") * nc + lax.axis_index("c")
        row0 = wid * rpt
        pltpu.sync_copy(
            i_hbm.at[pl.ds((row_base + row0) * s, rpt * s)], idx_v)

        bufs, gsems, osems = [buf0, buf1], [g0, g1], [o0, o1]
        gh = [None] * nchunk
        gh[0] = pltpu.async_copy(
            table_hbm.at[idx_v.at[pl.ds(0, w)]], bufs[0], gsems[0])
        if nchunk > 1:
            gh[1] = pltpu.async_copy(
                table_hbm.at[idx_v.at[pl.ds(w, w)]], bufs[1], gsems[1])
        for ci in range(nchunk):
            slot = ci % 2
            gh[ci].wait()
            hs = [
                pltpu.async_copy(
                    bufs[slot].at[pl.ds(j * s, s)],
                    o_hbm.at[row0 + ci * c + j],
                    osems[slot],
                )
                for j in range(c)
            ]
            for h in hs:
                h.wait()
            if ci + 2 < nchunk:
                gh[ci + 2] = pltpu.async_copy(
                    table_hbm.at[idx_v.at[pl.ds((ci + 2) * w, w)]],
                    bufs[slot],
                    gsems[slot],
                )

    return k(table, idx)


def kernel(node_ids, table):
    n, s = node_ids.shape
    d = table.shape[1]
    idx = node_ids.reshape(n * s).astype(jnp.int32)
    ch = n // _NCHUNKS
    parts = [
        _gather3d(table, idx, n, s, k * ch, ch) for k in range(_NCHUNKS)
    ]
    return jnp.concatenate([p * _SCALE for p in parts], axis=0)


# speedup vs baseline: 1.6945x; 1.6945x over previous
import functools
import math

import jax
import jax.numpy as jnp
from jax import lax
from jax.experimental import pallas as pl
from jax.experimental.pallas import tpu as pltpu
from jax.experimental.pallas import tpu_sc as plsc

_SCALE = math.sqrt(128.0)


def _scale_body(t_ref, o_ref):
    o_ref[...] = t_ref[...] * _SCALE


def _prescale(table):
    v, d = table.shape
    br = 10000
    return pl.pallas_call(
        _scale_body,
        out_shape=jax.ShapeDtypeStruct((v, d), table.dtype),
        grid=(v // br,),
        in_specs=[pl.BlockSpec((br, d), lambda i: (i, 0))],
        out_specs=pl.BlockSpec((br, d), lambda i: (i, 0)),
    )(table)


def _gather3d(table, idx, n, s):
    d = table.shape[1]
    nc, ns = 2, 16
    nw = nc * ns
    rpt = n // nw
    c = 8
    nchunk = rpt // c
    w = c * s

    mesh = plsc.VectorSubcoreMesh(core_axis_name="c", subcore_axis_name="s")

    @functools.partial(
        pl.kernel,
        out_type=jax.ShapeDtypeStruct((n, s, d), jnp.float32),
        mesh=mesh,
        scratch_types=[
            pltpu.VMEM((rpt * s,), jnp.int32),
            pltpu.VMEM((w, d), jnp.float32),
            pltpu.VMEM((w, d), jnp.float32),
            pltpu.SemaphoreType.DMA,
            pltpu.SemaphoreType.DMA,
            pltpu.SemaphoreType.DMA,
            pltpu.SemaphoreType.DMA,
        ],
    )
    def k(table_hbm, i_hbm, o_hbm, idx_v, buf0, buf1, g0, g1, o0, o1):
        wid = lax.axis_index("s") * nc + lax.axis_index("c")
        row0 = wid * rpt
        pltpu.sync_copy(i_hbm.at[pl.ds(row0 * s, rpt * s)], idx_v)

        bufs, gsems, osems = [buf0, buf1], [g0, g1], [o0, o1]
        gh = [None] * nchunk
        gh[0] = pltpu.async_copy(
            table_hbm.at[idx_v.at[pl.ds(0, w)]], bufs[0], gsems[0])
        if nchunk > 1:
            gh[1] = pltpu.async_copy(
                table_hbm.at[idx_v.at[pl.ds(w, w)]], bufs[1], gsems[1])
        for ci in range(nchunk):
            slot = ci % 2
            gh[ci].wait()
            hs = [
                pltpu.async_copy(
                    bufs[slot].at[pl.ds(j * s, s)],
                    o_hbm.at[row0 + ci * c + j],
                    osems[slot],
                )
                for j in range(c)
            ]
            for h in hs:
                h.wait()
            if ci + 2 < nchunk:
                gh[ci + 2] = pltpu.async_copy(
                    table_hbm.at[idx_v.at[pl.ds((ci + 2) * w, w)]],
                    bufs[slot],
                    gsems[slot],
                )

    return k(table, idx)


def kernel(node_ids, table):
    n, s = node_ids.shape
    scaled = _prescale(table)
    idx = node_ids.reshape(n * s).astype(jnp.int32)
    return _gather3d(scaled, idx, n, s)


# speedup vs baseline: 1.7196x; 1.0148x over previous
import functools
import math

import jax
import jax.numpy as jnp
from jax import lax
from jax.experimental import pallas as pl
from jax.experimental.pallas import tpu as pltpu
from jax.experimental.pallas import tpu_sc as plsc

_SCALE = math.sqrt(128.0)


def _scale_body(t_ref, o_ref):
    o_ref[...] = t_ref[...] * _SCALE


def _prescale(table):
    v, d = table.shape
    br = 10000
    return pl.pallas_call(
        _scale_body,
        out_shape=jax.ShapeDtypeStruct((v, d), table.dtype),
        grid=(v // br,),
        in_specs=[pl.BlockSpec((br, d), lambda i: (i, 0))],
        out_specs=pl.BlockSpec((br, d), lambda i: (i, 0)),
    )(table)


def _gather3d(table, idx, n, s):
    d = table.shape[1]
    nc, ns = 2, 16
    nw = nc * ns
    rpt = n // nw
    c = 8
    nchunk = rpt // c
    w = c * s

    mesh = plsc.VectorSubcoreMesh(core_axis_name="c", subcore_axis_name="s")

    @functools.partial(
        pl.kernel,
        out_type=jax.ShapeDtypeStruct((n, s, d), jnp.float32),
        mesh=mesh,
        scratch_types=[
            pltpu.VMEM((rpt * s,), jnp.int32),
            pltpu.VMEM((w, d), jnp.float32),
            pltpu.VMEM((w, d), jnp.float32),
            pltpu.SemaphoreType.DMA,
            pltpu.SemaphoreType.DMA,
            pltpu.SemaphoreType.DMA,
            pltpu.SemaphoreType.DMA,
        ],
    )
    def k(table_hbm, i_hbm, o_hbm, idx_v, buf0, buf1, g0, g1, o0, o1):
        wid = lax.axis_index("s") * nc + lax.axis_index("c")
        row0 = wid * rpt
        pltpu.sync_copy(i_hbm.at[pl.ds(row0 * s, rpt * s)], idx_v)

        bufs, gsems, osems = [buf0, buf1], [g0, g1], [o0, o1]
        gh = [None] * nchunk
        gh[0] = pltpu.async_copy(
            table_hbm.at[idx_v.at[pl.ds(0, w)]], bufs[0], gsems[0])
        if nchunk > 1:
            gh[1] = pltpu.async_copy(
                table_hbm.at[idx_v.at[pl.ds(w, w)]], bufs[1], gsems[1])
        for ci in range(nchunk):
            slot = ci % 2
            gh[ci].wait()
            pltpu.async_copy(
                bufs[slot].reshape(c, s, d),
                o_hbm.at[pl.ds(row0 + ci * c, c)],
                osems[slot],
            ).wait()
            if ci + 2 < nchunk:
                gh[ci + 2] = pltpu.async_copy(
                    table_hbm.at[idx_v.at[pl.ds((ci + 2) * w, w)]],
                    bufs[slot],
                    gsems[slot],
                )

    return k(table, idx)


def kernel(node_ids, table):
    n, s = node_ids.shape
    scaled = _prescale(table)
    idx = node_ids.reshape(n * s).astype(jnp.int32)
    return _gather3d(scaled, idx, n, s)
